# baseline (speedup 1.0000x reference)
import functools

import jax
import jax.numpy as jnp
from jax.experimental import pallas as pl

_NP = 98
_NPP = 104
_HW = 64
_BB = 32


def _gcn_body(x_ref, a0_ref, a1_ref, bcol_ref, w0a_ref, w0b_ref, b0_ref,
              w1_ref, b1_ref, w2_ref, b2_ref, w3_ref, b3_ref,
              cs_ref, cd_ref, cu_ref, out_ref):
    a0 = a0_ref[...]
    a1 = a1_ref[...]
    bcol = bcol_ref[...]
    chunks01 = []
    chunks23 = []
    for g in range(_BB // 4):
        xl = jnp.concatenate(
            [x_ref[(4 * g + i) * 128:(4 * g + i + 1) * 128, 0:64]
             for i in range(4)], axis=1)
        xr = jnp.concatenate(
            [x_ref[(4 * g + i) * 128:(4 * g + i + 1) * 128, 64:128]
             for i in range(4)], axis=1)
        blk = (jax.lax.dot_general(
                   a0, xl, (((1,), (0,)), ((), ())),
                   preferred_element_type=jnp.float32)
               + jax.lax.dot_general(
                   a1, xr, (((1,), (0,)), ((), ())),
                   preferred_element_type=jnp.float32) + bcol)
        blk = jnp.maximum(blk, 0.0)
        for i in range(4):
            cols = blk[:, 64 * i:64 * (i + 1)]
            chunks01.append(
                jnp.concatenate([cols[0:_NPP], cols[_NPP:2 * _NPP]], axis=1))
            chunks23.append(
                jnp.concatenate([cols[2 * _NPP:3 * _NPP],
                                 cols[3 * _NPP:4 * _NPP]], axis=1))
    h01 = jnp.concatenate(chunks01, axis=0)
    h23 = jnp.concatenate(chunks23, axis=0)

    cs = cs_ref[...]
    cd = cd_ref[...]
    cu = cu_ref[...]

    def prop(h):
        z = jnp.zeros((1, h.shape[1]), h.dtype)
        up = jnp.concatenate([z, h[:-1]], axis=0)
        dn = jnp.concatenate([h[1:], z], axis=0)
        return cs * up + cd * h + cu * dn

    def mm(a, b):
        return jax.lax.dot_general(
            a, b, (((1,), (0,)), ((), ())),
            preferred_element_type=jnp.float32)

    y = mm(h01, w0a_ref[...]) + mm(h23, w0b_ref[...])
    h = jnp.maximum(prop(y) + b0_ref[...], 0.0)
    h = jnp.maximum(prop(mm(h, w1_ref[...])) + b1_ref[...], 0.0)
    h = jnp.maximum(prop(mm(h, w2_ref[...])) + b2_ref[...], 0.0)
    h3 = prop(mm(h, w3_ref[...])) + b3_ref[...]
    sub = jax.lax.broadcasted_iota(jnp.int32, (_NPP, 2 * _NPP), 0)
    lane = jax.lax.broadcasted_iota(jnp.int32, (_NPP, 2 * _NPP), 1)
    mask = jnp.where(lane // 2 == sub, 1.0, 0.0).astype(jnp.float32)
    h3m = h3.reshape(_BB, _NPP, 2 * _NPP) * mask[None]
    out_ref[...] = jnp.sum(h3m, axis=1)


@jax.jit
def kernel(x, W_map, b_map, A, W0, b0, W1, b1, W2, b2, W3, b3):
    B, C, H, W = x.shape
    hw = H * W
    x2 = x.reshape(B * (C // 2), 2 * hw)

    wre = jnp.zeros((4 * _NPP, C), x.dtype)
    bcol = jnp.zeros((4 * _NPP, 1), x.dtype)
    for c4 in range(4):
        wre = wre.at[c4 * _NPP:c4 * _NPP + _NP].set(W_map[c4::4])
        bcol = bcol.at[c4 * _NPP:c4 * _NPP + _NP, 0].set(b_map[c4::4])

    cd98 = jnp.diagonal(A)
    cs98 = jnp.zeros((_NP,), A.dtype).at[1:].set(jnp.diagonal(A, offset=-1))
    cu98 = jnp.zeros((_NP,), A.dtype).at[:-1].set(jnp.diagonal(A, offset=1))

    def expand(c98):
        c = jnp.zeros((_NPP,), A.dtype).at[:_NP].set(c98)
        return jnp.tile(c, (_BB,)).reshape(_BB * _NPP, 1)

    cs = expand(cs98)
    cd = expand(cd98)
    cu = expand(cu98)

    a0 = wre[:, 0::2]
    a1 = wre[:, 1::2]
    w0a = W0[:128]
    w0b = W0[128:]
    b0r = b0.reshape(1, -1)
    b1r = b1.reshape(1, -1)
    b2r = b2.reshape(1, -1)
    w3t = jnp.tile(W3, (1, _NPP))
    b3t = jnp.tile(b3.reshape(1, 2), (1, _NPP))

    full = lambda shape: pl.BlockSpec(shape, lambda i: (0,) * len(shape))
    out = pl.pallas_call(
        _gcn_body,
        grid=(B // _BB,),
        in_specs=[
            pl.BlockSpec((_BB * 128, 128), lambda i: (i, 0)),
            full((4 * _NPP, 128)),
            full((4 * _NPP, 128)),
            full((4 * _NPP, 1)),
            full((128, 128)),
            full((128, 128)),
            full((1, 128)),
            full((128, 128)),
            full((1, 128)),
            full((128, 128)),
            full((1, 128)),
            full((128, 2 * _NPP)),
            full((1, 2 * _NPP)),
            full((_BB * _NPP, 1)),
            full((_BB * _NPP, 1)),
            full((_BB * _NPP, 1)),
        ],
        out_specs=pl.BlockSpec((_BB, _NPP * 2), lambda i: (i, 0)),
        out_shape=jax.ShapeDtypeStruct((B, _NPP * 2), jnp.float32),
    )(x2, a0, a1, bcol, w0a, w0b, b0r, W1, b1r, W2, b2r, w3t, b3t, cs, cd, cu)

    return out[:, :2 * _NP]


# speedup vs baseline: 4.7195x; 4.7195x over previous
import functools

import jax
import jax.numpy as jnp
from jax.experimental import pallas as pl

_NP = 98
_NPP = 104
_HW = 64
_BB = 32


def _gcn_body(x_ref, wre_ref, bcol_ref, w0a_ref, w0b_ref, b0_ref,
              w1_ref, b1_ref, w2_ref, b2_ref, w3_ref, b3_ref,
              cs_ref, cd_ref, cu_ref, out_ref):
    wre = wre_ref[...]
    bcol = bcol_ref[...]
    chunks01 = []
    chunks23 = []
    for g in range(_BB // 4):
        xg = x_ref[g * 256:(g + 1) * 256, :]
        blk = jax.lax.dot_general(
            wre, xg, (((1,), (1,)), ((), ())),
            preferred_element_type=jnp.float32) + bcol
        blk = jnp.maximum(blk, 0.0)
        for i in range(4):
            cols = blk[:, 64 * i:64 * (i + 1)]
            chunks01.append(
                jnp.concatenate([cols[0:_NPP], cols[_NPP:2 * _NPP]], axis=1))
            chunks23.append(
                jnp.concatenate([cols[2 * _NPP:3 * _NPP],
                                 cols[3 * _NPP:4 * _NPP]], axis=1))
    h01 = jnp.concatenate(chunks01, axis=0)
    h23 = jnp.concatenate(chunks23, axis=0)

    cs = cs_ref[...]
    cd = cd_ref[...]
    cu = cu_ref[...]

    def prop(h):
        z = jnp.zeros((1, h.shape[1]), h.dtype)
        up = jnp.concatenate([z, h[:-1]], axis=0)
        dn = jnp.concatenate([h[1:], z], axis=0)
        return cs * up + cd * h + cu * dn

    def mm(a, b):
        return jax.lax.dot_general(
            a, b, (((1,), (0,)), ((), ())),
            preferred_element_type=jnp.float32)

    y = mm(h01, w0a_ref[...]) + mm(h23, w0b_ref[...])
    h = jnp.maximum(prop(y) + b0_ref[...], 0.0)
    h = jnp.maximum(prop(mm(h, w1_ref[...])) + b1_ref[...], 0.0)
    h = jnp.maximum(prop(mm(h, w2_ref[...])) + b2_ref[...], 0.0)
    h3 = prop(mm(h, w3_ref[...])) + b3_ref[...]
    sub = jax.lax.broadcasted_iota(jnp.int32, (_NPP, 2 * _NPP), 0)
    lane = jax.lax.broadcasted_iota(jnp.int32, (_NPP, 2 * _NPP), 1)
    mask = jnp.where(lane // 2 == sub, 1.0, 0.0).astype(jnp.float32)
    h3m = h3.reshape(_BB, _NPP, 2 * _NPP) * mask[None]
    out_ref[...] = jnp.sum(h3m, axis=1)


@jax.jit
def kernel(x, W_map, b_map, A, W0, b0, W1, b1, W2, b2, W3, b3):
    B, C, H, W = x.shape
    hw = H * W
    xt = x.transpose(0, 2, 3, 1).reshape(B * hw, C)

    wre = jnp.zeros((4 * _NPP, C), x.dtype)
    bcol = jnp.zeros((4 * _NPP, 1), x.dtype)
    for c4 in range(4):
        wre = wre.at[c4 * _NPP:c4 * _NPP + _NP].set(W_map[c4::4])
        bcol = bcol.at[c4 * _NPP:c4 * _NPP + _NP, 0].set(b_map[c4::4])

    cd98 = jnp.diagonal(A)
    cs98 = jnp.zeros((_NP,), A.dtype).at[1:].set(jnp.diagonal(A, offset=-1))
    cu98 = jnp.zeros((_NP,), A.dtype).at[:-1].set(jnp.diagonal(A, offset=1))

    def expand(c98):
        c = jnp.zeros((_NPP,), A.dtype).at[:_NP].set(c98)
        return jnp.tile(c, (_BB,)).reshape(_BB * _NPP, 1)

    cs = expand(cs98)
    cd = expand(cd98)
    cu = expand(cu98)

    w0a = W0[:128]
    w0b = W0[128:]
    b0r = b0.reshape(1, -1)
    b1r = b1.reshape(1, -1)
    b2r = b2.reshape(1, -1)
    w3t = jnp.tile(W3, (1, _NPP))
    b3t = jnp.tile(b3.reshape(1, 2), (1, _NPP))

    full = lambda shape: pl.BlockSpec(shape, lambda i: (0,) * len(shape))
    out = pl.pallas_call(
        _gcn_body,
        grid=(B // _BB,),
        in_specs=[
            pl.BlockSpec((_BB * hw, C), lambda i: (i, 0)),
            full((4 * _NPP, C)),
            full((4 * _NPP, 1)),
            full((128, 128)),
            full((128, 128)),
            full((1, 128)),
            full((128, 128)),
            full((1, 128)),
            full((128, 128)),
            full((1, 128)),
            full((128, 2 * _NPP)),
            full((1, 2 * _NPP)),
            full((_BB * _NPP, 1)),
            full((_BB * _NPP, 1)),
            full((_BB * _NPP, 1)),
        ],
        out_specs=pl.BlockSpec((_BB, _NPP * 2), lambda i: (i, 0)),
        out_shape=jax.ShapeDtypeStruct((B, _NPP * 2), jnp.float32),
    )(xt, wre, bcol, w0a, w0b, b0r, W1, b1r, W2, b2r, w3t, b3t, cs, cd, cu)

    return out[:, :2 * _NP]
